# Initial kernel scaffold; baseline (speedup 1.0000x reference)
import jax
import jax.numpy as jnp
from jax.experimental import pallas as pl


def kernel(anchors, n, d):
    raise NotImplementedError("write your pallas kernel here")


# speedup vs baseline: 18.1799x; 18.1799x over previous
import functools

import jax
import jax.numpy as jnp
from jax.experimental import pallas as pl

_BETA = 1.0
_TEMPERATURE = 0.07
_K = 8
_BIAS_VAL = _BETA / _TEMPERATURE


def _bias_block_kernel(c_ref, mT_ref, arows_ref, out_ref, *, R, M):
    c = c_ref[0:1, :]
    gx = -2.0 * mT_ref[0:1, :]
    gy = -2.0 * mT_ref[1:2, :]
    gz = -2.0 * mT_ref[2:3, :]
    a0 = arows_ref[:, 0:1]
    a1 = arows_ref[:, 1:2]
    a2 = arows_ref[:, 2:3]
    S = c + a0 * gx + a1 * gy + a2 * gz

    iota = jax.lax.broadcasted_iota(jnp.int32, (R, M), 1)
    acc = jnp.zeros((R, M), dtype=jnp.bool_)
    for _ in range(_K):
        m = jnp.min(S, axis=1, keepdims=True)
        cand = jnp.where(S == m, iota, M)
        j = jnp.min(cand, axis=1, keepdims=True)
        hit = iota == j
        acc = jnp.logical_or(acc, hit)
        S = jnp.where(hit, jnp.inf, S)
    out_ref[...] = jnp.where(acc, _BIAS_VAL, 0.0).astype(jnp.float32)


def kernel(anchors, n, d):
    B, M, _ = anchors.shape
    R = 128
    n_hat = n / (jnp.linalg.norm(n, axis=-1, keepdims=True) + 1e-8)
    s = jnp.einsum('bmc,bc->bm', anchors, n_hat) + d
    mirrored = anchors - 2.0 * s[..., None] * n_hat[:, None, :]
    mT = jnp.swapaxes(mirrored, 1, 2)
    c = jnp.sum(mirrored * mirrored, axis=-1)[:, None, :]

    body = functools.partial(_bias_block_kernel, R=R, M=M)
    out = pl.pallas_call(
        body,
        grid=(B, M // R),
        in_specs=[
            pl.BlockSpec((None, 1, M), lambda b, r: (b, 0, 0)),
            pl.BlockSpec((None, 3, M), lambda b, r: (b, 0, 0)),
            pl.BlockSpec((None, R, 3), lambda b, r: (b, r, 0)),
        ],
        out_specs=pl.BlockSpec((None, R, M), lambda b, r: (b, r, 0)),
        out_shape=jax.ShapeDtypeStruct((B, M, M), jnp.float32),
    )(c, mT, anchors)
    return out


# speedup vs baseline: 45.6213x; 2.5094x over previous
import functools

import jax
import jax.numpy as jnp
from jax.experimental import pallas as pl

_BETA = 1.0
_TEMPERATURE = 0.07
_K = 8
_BIAS_VAL = _BETA / _TEMPERATURE


def _bias_block_kernel(c_ref, mT_ref, arows_ref, out_ref, *, R, M):
    c = c_ref[0:1, :]
    gx = -2.0 * mT_ref[0:1, :]
    gy = -2.0 * mT_ref[1:2, :]
    gz = -2.0 * mT_ref[2:3, :]
    a0 = arows_ref[:, 0:1]
    a1 = arows_ref[:, 1:2]
    a2 = arows_ref[:, 2:3]
    S = c + a0 * gx + a1 * gy + a2 * gz

    W = S
    tau = jnp.min(W, axis=1, keepdims=True)
    for _ in range(_K - 1):
        W = jnp.where(W == tau, jnp.inf, W)
        tau = jnp.min(W, axis=1, keepdims=True)
    sel = S <= tau
    out_ref[...] = jnp.where(sel, _BIAS_VAL, 0.0).astype(jnp.float32)

    counts = jnp.sum(sel.astype(jnp.int32), axis=1)
    bad = jnp.any(counts != _K)

    @pl.when(bad)
    def _exact_fallback():
        iota = jax.lax.broadcasted_iota(jnp.int32, (R, M), 1)
        Sf = S
        acc = jnp.zeros((R, M), dtype=jnp.bool_)
        for _ in range(_K):
            m = jnp.min(Sf, axis=1, keepdims=True)
            cand = jnp.where(Sf == m, iota, M)
            j = jnp.min(cand, axis=1, keepdims=True)
            hit = iota == j
            acc = jnp.logical_or(acc, hit)
            Sf = jnp.where(hit, jnp.inf, Sf)
        out_ref[...] = jnp.where(acc, _BIAS_VAL, 0.0).astype(jnp.float32)


def kernel(anchors, n, d):
    B, M, _ = anchors.shape
    R = 128
    n_hat = n / (jnp.linalg.norm(n, axis=-1, keepdims=True) + 1e-8)
    s = jnp.einsum('bmc,bc->bm', anchors, n_hat) + d
    mirrored = anchors - 2.0 * s[..., None] * n_hat[:, None, :]
    mT = jnp.swapaxes(mirrored, 1, 2)
    c = jnp.sum(mirrored * mirrored, axis=-1)[:, None, :]

    body = functools.partial(_bias_block_kernel, R=R, M=M)
    out = pl.pallas_call(
        body,
        grid=(B, M // R),
        in_specs=[
            pl.BlockSpec((None, 1, M), lambda b, r: (b, 0, 0)),
            pl.BlockSpec((None, 3, M), lambda b, r: (b, 0, 0)),
            pl.BlockSpec((None, R, 3), lambda b, r: (b, r, 0)),
        ],
        out_specs=pl.BlockSpec((None, R, M), lambda b, r: (b, r, 0)),
        out_shape=jax.ShapeDtypeStruct((B, M, M), jnp.float32),
    )(c, mT, anchors)
    return out


# speedup vs baseline: 58.2781x; 1.2774x over previous
import functools

import jax
import jax.numpy as jnp
from jax.experimental import pallas as pl

_BETA = 1.0
_TEMPERATURE = 0.07
_K = 8
_BIAS_VAL = _BETA / _TEMPERATURE


def _bias_block_kernel(c_ref, mT_ref, arows_ref, out_ref, *, R, M):
    c = c_ref[0:1, :]
    gx = -2.0 * mT_ref[0:1, :]
    gy = -2.0 * mT_ref[1:2, :]
    gz = -2.0 * mT_ref[2:3, :]
    a0 = arows_ref[:, 0:1]
    a1 = arows_ref[:, 1:2]
    a2 = arows_ref[:, 2:3]
    S = c + a0 * gx + a1 * gy + a2 * gz

    NCOL = M // 128
    cols = [S[:, k * 128:(k + 1) * 128] for k in range(NCOL)]
    INF = jnp.float32(jnp.inf)

    m0 = cols[0]
    m1 = jnp.full_like(m0, INF)
    m2 = m1
    m3 = m1
    for k in range(1, NCOL):
        v = cols[k]
        t = jnp.maximum(m0, v)
        m0 = jnp.minimum(m0, v)
        v = t
        t = jnp.maximum(m1, v)
        m1 = jnp.minimum(m1, v)
        v = t
        t = jnp.maximum(m2, v)
        m2 = jnp.minimum(m2, v)
        m3 = jnp.minimum(m3, t)

    cur = m0
    cnt = jnp.zeros_like(m0, dtype=jnp.int32)
    tau = None
    for _ in range(_K):
        tau = jnp.min(cur, axis=1, keepdims=True)
        hit = cur == tau
        cnt = cnt + hit.astype(jnp.int32)
        nxt = jnp.where(cnt == 1, m1,
                        jnp.where(cnt == 2, m2,
                                  jnp.where(cnt == 3, m3, INF)))
        cur = jnp.where(hit, nxt, cur)

    count = None
    for k in range(NCOL):
        selk = cols[k] <= tau
        out_ref[:, k * 128:(k + 1) * 128] = jnp.where(
            selk, _BIAS_VAL, 0.0).astype(jnp.float32)
        ck = selk.astype(jnp.int32)
        count = ck if count is None else count + ck
    counts = jnp.sum(count, axis=1)
    bad = jnp.any(counts != _K)

    @pl.when(bad)
    def _exact_fallback():
        iota = jax.lax.broadcasted_iota(jnp.int32, (R, M), 1)
        Sf = S
        acc = jnp.zeros((R, M), dtype=jnp.bool_)
        for _ in range(_K):
            m = jnp.min(Sf, axis=1, keepdims=True)
            cand = jnp.where(Sf == m, iota, M)
            j = jnp.min(cand, axis=1, keepdims=True)
            hit = iota == j
            acc = jnp.logical_or(acc, hit)
            Sf = jnp.where(hit, jnp.inf, Sf)
        out_ref[...] = jnp.where(acc, _BIAS_VAL, 0.0).astype(jnp.float32)


def kernel(anchors, n, d):
    B, M, _ = anchors.shape
    R = 128
    n_hat = n / (jnp.linalg.norm(n, axis=-1, keepdims=True) + 1e-8)
    s = jnp.einsum('bmc,bc->bm', anchors, n_hat) + d
    mirrored = anchors - 2.0 * s[..., None] * n_hat[:, None, :]
    mT = jnp.swapaxes(mirrored, 1, 2)
    c = jnp.sum(mirrored * mirrored, axis=-1)[:, None, :]

    body = functools.partial(_bias_block_kernel, R=R, M=M)
    out = pl.pallas_call(
        body,
        grid=(B, M // R),
        in_specs=[
            pl.BlockSpec((None, 1, M), lambda b, r: (b, 0, 0)),
            pl.BlockSpec((None, 3, M), lambda b, r: (b, 0, 0)),
            pl.BlockSpec((None, R, 3), lambda b, r: (b, r, 0)),
        ],
        out_specs=pl.BlockSpec((None, R, M), lambda b, r: (b, r, 0)),
        out_shape=jax.ShapeDtypeStruct((B, M, M), jnp.float32),
    )(c, mT, anchors)
    return out


# speedup vs baseline: 71.5083x; 1.2270x over previous
import functools

import jax
import jax.numpy as jnp
from jax.experimental import pallas as pl

_BETA = 1.0
_TEMPERATURE = 0.07
_K = 8
_BIAS_VAL = _BETA / _TEMPERATURE


def _bias_block_kernel(c_ref, mT_ref, arows_ref, out_ref, *, R, M):
    c = c_ref[0:1, :]
    gx = -2.0 * mT_ref[0:1, :]
    gy = -2.0 * mT_ref[1:2, :]
    gz = -2.0 * mT_ref[2:3, :]
    a0 = arows_ref[:, 0:1]
    a1 = arows_ref[:, 1:2]
    a2 = arows_ref[:, 2:3]
    S = c + a0 * gx + a1 * gy + a2 * gz

    NCOL = M // 128
    cols = [S[:, k * 128:(k + 1) * 128] for k in range(NCOL)]
    INF = jnp.float32(jnp.inf)

    m0 = cols[0]
    m1 = jnp.full_like(m0, INF)
    m2 = m1
    m3 = m1
    for k in range(1, NCOL):
        v = cols[k]
        t = jnp.maximum(m0, v)
        m0 = jnp.minimum(m0, v)
        v = t
        t = jnp.maximum(m1, v)
        m1 = jnp.minimum(m1, v)
        v = t
        t = jnp.maximum(m2, v)
        m2 = jnp.minimum(m2, v)
        m3 = jnp.minimum(m3, t)

    cur = m0
    cnt = jnp.zeros_like(m0, dtype=jnp.int32)
    tau = None
    for _ in range(_K):
        tau = jnp.min(cur, axis=1, keepdims=True)
        hit = cur == tau
        cnt = cnt + hit.astype(jnp.int32)
        nxt = jnp.where(cnt == 1, m1,
                        jnp.where(cnt == 2, m2,
                                  jnp.where(cnt == 3, m3, INF)))
        cur = jnp.where(hit, nxt, cur)

    count = None
    for k in range(NCOL):
        selk = cols[k] <= tau
        out_ref[:, k * 128:(k + 1) * 128] = jnp.where(
            selk, _BIAS_VAL, 0.0).astype(jnp.float32)
        ck = selk.astype(jnp.int32)
        count = ck if count is None else count + ck
    counts = jnp.sum(count, axis=1)
    bad = jnp.any(counts != _K)

    @pl.when(bad)
    def _exact_fallback():
        iota = jax.lax.broadcasted_iota(jnp.int32, (R, M), 1)
        Sf = S
        acc = jnp.zeros((R, M), dtype=jnp.bool_)
        for _ in range(_K):
            m = jnp.min(Sf, axis=1, keepdims=True)
            cand = jnp.where(Sf == m, iota, M)
            j = jnp.min(cand, axis=1, keepdims=True)
            hit = iota == j
            acc = jnp.logical_or(acc, hit)
            Sf = jnp.where(hit, jnp.inf, Sf)
        out_ref[...] = jnp.where(acc, _BIAS_VAL, 0.0).astype(jnp.float32)


def kernel(anchors, n, d):
    B, M, _ = anchors.shape
    R = 256
    n_hat = n / (jnp.linalg.norm(n, axis=-1, keepdims=True) + 1e-8)
    s = jnp.einsum('bmc,bc->bm', anchors, n_hat) + d
    mirrored = anchors - 2.0 * s[..., None] * n_hat[:, None, :]
    mT = jnp.swapaxes(mirrored, 1, 2)
    c = jnp.sum(mirrored * mirrored, axis=-1)[:, None, :]

    body = functools.partial(_bias_block_kernel, R=R, M=M)
    out = pl.pallas_call(
        body,
        grid=(B, M // R),
        in_specs=[
            pl.BlockSpec((None, 1, M), lambda b, r: (b, 0, 0)),
            pl.BlockSpec((None, 3, M), lambda b, r: (b, 0, 0)),
            pl.BlockSpec((None, R, 3), lambda b, r: (b, r, 0)),
        ],
        out_specs=pl.BlockSpec((None, R, M), lambda b, r: (b, r, 0)),
        out_shape=jax.ShapeDtypeStruct((B, M, M), jnp.float32),
    )(c, mT, anchors)
    return out


# speedup vs baseline: 79.1194x; 1.1064x over previous
import functools

import jax
import jax.numpy as jnp
from jax.experimental import pallas as pl

_BETA = 1.0
_TEMPERATURE = 0.07
_K = 8
_BIAS_VAL = _BETA / _TEMPERATURE


def _bias_block_kernel(c_ref, mT_ref, arows_ref, out_ref, *, R, M):
    c = c_ref[0:1, :]
    gx = -2.0 * mT_ref[0:1, :]
    gy = -2.0 * mT_ref[1:2, :]
    gz = -2.0 * mT_ref[2:3, :]
    a0 = arows_ref[:, 0:1]
    a1 = arows_ref[:, 1:2]
    a2 = arows_ref[:, 2:3]
    S = c + a0 * gx + a1 * gy + a2 * gz

    NCOL = M // 128
    cols = [S[:, k * 128:(k + 1) * 128] for k in range(NCOL)]
    INF = jnp.float32(jnp.inf)

    m0 = cols[0]
    m1 = jnp.full_like(m0, INF)
    m2 = m1
    m3 = m1
    for k in range(1, NCOL):
        v = cols[k]
        t = jnp.maximum(m0, v)
        m0 = jnp.minimum(m0, v)
        v = t
        t = jnp.maximum(m1, v)
        m1 = jnp.minimum(m1, v)
        v = t
        t = jnp.maximum(m2, v)
        m2 = jnp.minimum(m2, v)
        m3 = jnp.minimum(m3, t)

    cur = m0
    cnt = jnp.zeros_like(m0, dtype=jnp.int32)
    tau = None
    for _ in range(_K):
        tau = jnp.min(cur, axis=1, keepdims=True)
        hit = cur == tau
        cnt = cnt + hit.astype(jnp.int32)
        nxt = jnp.where(cnt == 1, m1,
                        jnp.where(cnt == 2, m2,
                                  jnp.where(cnt == 3, m3, INF)))
        cur = jnp.where(hit, nxt, cur)

    count = None
    for k in range(NCOL):
        selk = cols[k] <= tau
        out_ref[:, k * 128:(k + 1) * 128] = jnp.where(
            selk, _BIAS_VAL, 0.0).astype(jnp.float32)
        ck = selk.astype(jnp.int32)
        count = ck if count is None else count + ck
    counts = jnp.sum(count, axis=1)
    bad = jnp.any(counts != _K)

    @pl.when(bad)
    def _exact_fallback():
        iota = jax.lax.broadcasted_iota(jnp.int32, (R, M), 1)
        Sf = S
        acc = jnp.zeros((R, M), dtype=jnp.bool_)
        for _ in range(_K):
            m = jnp.min(Sf, axis=1, keepdims=True)
            cand = jnp.where(Sf == m, iota, M)
            j = jnp.min(cand, axis=1, keepdims=True)
            hit = iota == j
            acc = jnp.logical_or(acc, hit)
            Sf = jnp.where(hit, jnp.inf, Sf)
        out_ref[...] = jnp.where(acc, _BIAS_VAL, 0.0).astype(jnp.float32)


def kernel(anchors, n, d):
    B, M, _ = anchors.shape
    R = 512
    n_hat = n / (jnp.linalg.norm(n, axis=-1, keepdims=True) + 1e-8)
    s = jnp.einsum('bmc,bc->bm', anchors, n_hat) + d
    mirrored = anchors - 2.0 * s[..., None] * n_hat[:, None, :]
    mT = jnp.swapaxes(mirrored, 1, 2)
    c = jnp.sum(mirrored * mirrored, axis=-1)[:, None, :]

    body = functools.partial(_bias_block_kernel, R=R, M=M)
    out = pl.pallas_call(
        body,
        grid=(B, M // R),
        in_specs=[
            pl.BlockSpec((None, 1, M), lambda b, r: (b, 0, 0)),
            pl.BlockSpec((None, 3, M), lambda b, r: (b, 0, 0)),
            pl.BlockSpec((None, R, 3), lambda b, r: (b, r, 0)),
        ],
        out_specs=pl.BlockSpec((None, R, M), lambda b, r: (b, r, 0)),
        out_shape=jax.ShapeDtypeStruct((B, M, M), jnp.float32),
    )(c, mT, anchors)
    return out
